# baseline (speedup 1.0000x reference)
import functools

import jax
import jax.numpy as jnp
from jax import lax
from jax.experimental import pallas as pl


def _cdiv(a, b):
    return (a + b - 1) // b


def _dense_body(x_ref, w_ref, b_ref, g_ref, bb_ref, o_ref, *, ln, act):
    x = x_ref[...]
    y = jnp.dot(x, w_ref[...], preferred_element_type=jnp.float32) + b_ref[...]
    if ln:
        mu = jnp.mean(y, axis=-1, keepdims=True)
        var = jnp.mean((y - mu) ** 2, axis=-1, keepdims=True)
        y = (y - mu) * lax.rsqrt(var + 1e-5) * g_ref[...] + bb_ref[...]
    if act:
        y = y * jax.nn.sigmoid(y)
    o_ref[...] = y


@functools.partial(jax.jit, static_argnames=("ln", "act", "blk"))
def _dense(x, W, b, g, bb, ln=True, act=True, blk=1024):
    n, fin = x.shape
    fout = W.shape[1]
    npad = _cdiv(n, blk) * blk
    if npad != n:
        x = jnp.pad(x, ((0, npad - n), (0, 0)))
    grid = (npad // blk,)
    out = pl.pallas_call(
        functools.partial(_dense_body, ln=ln, act=act),
        grid=grid,
        in_specs=[
            pl.BlockSpec((blk, fin), lambda i: (i, 0)),
            pl.BlockSpec((fin, fout), lambda i: (0, 0)),
            pl.BlockSpec((fout,), lambda i: (0,)),
            pl.BlockSpec((fout,), lambda i: (0,)),
            pl.BlockSpec((fout,), lambda i: (0,)),
        ],
        out_specs=pl.BlockSpec((blk, fout), lambda i: (i, 0)),
        out_shape=jax.ShapeDtypeStruct((npad, fout), jnp.float32),
    )(x, W, b, g, bb)
    return out[:n]


def _mlp(p, x):
    return _dense(x, p["lin"]["W"], p["lin"]["b"], p["ln"]["g"], p["ln"]["b"],
                  ln=True, act=True)


def _lin_pl(p, x):
    z = jnp.zeros((p["W"].shape[1],), jnp.float32)
    o = jnp.ones((p["W"].shape[1],), jnp.float32)
    return _dense(x, p["W"], p["b"], o, z, ln=False, act=False)


def _rbf(d, vmin, vmax, bins):
    centers = jnp.linspace(vmin, vmax, bins)
    gamma = 1.0 / (centers[1] - centers[0])
    return jnp.exp(-gamma * (d[:, None] - centers[None, :]) ** 2)


def _silu(x):
    return x * jax.nn.sigmoid(x)


def _ln(p, x):
    mu = x.mean(-1, keepdims=True)
    var = x.var(-1, keepdims=True)
    return (x - mu) / jnp.sqrt(var + 1e-5) * p["g"] + p["b"]


def _eggc(p, src, dst, n, h, e):
    m = _lin_pl(p["src_gate"], h)[src] + _lin_pl(p["dst_gate"], h)[dst] \
        + _lin_pl(p["edge_gate"], e)
    sigma = jax.nn.sigmoid(m)
    Bh = _lin_pl(p["dst_update"], h)
    sum_sigma_h = jax.ops.segment_sum(Bh[src] * sigma, dst, num_segments=n)
    sum_sigma = jax.ops.segment_sum(sigma, dst, num_segments=n)
    hagg = sum_sigma_h / (sum_sigma + 1e-6)
    x = _silu(_ln(p["bn_nodes"], _lin_pl(p["src_update"], h) + hagg))
    y = _silu(_ln(p["bn_edges"], m))
    return h + x, e + y


def kernel(x, r, edge_index, lg_edge_index, params):
    src, dst = edge_index[0], edge_index[1]
    lsrc, ldst = lg_edge_index[0], lg_edge_index[1]
    n = x.shape[0]
    n_edges = r.shape[0]
    h = _mlp(params["atom_embedding"], x)
    bondlength = jnp.linalg.norm(r, axis=1)
    y = _rbf(bondlength, 0.0, 8.0, 80)
    y = _mlp(params["edge_embedding"][0], y)
    y = _mlp(params["edge_embedding"][1], y)
    r1 = -r[lsrc]
    r2 = r[ldst]
    cos = (r1 * r2).sum(axis=1) / (jnp.linalg.norm(r1, axis=1) * jnp.linalg.norm(r2, axis=1))
    z = _rbf(cos, -1.0, 1.0, 40)
    z = _mlp(params["angle_embedding"][0], z)
    z = _mlp(params["angle_embedding"][1], z)
    for lp in params["alignn_layers"]:
        h, m = _eggc(lp["node_update"], src, dst, n, h, y)
        y, z = _eggc(lp["edge_update"], lsrc, ldst, n_edges, m, z)
    for gp in params["gcn_layers"]:
        h, y = _eggc(gp, src, dst, n, h, y)
    hg = h.mean(axis=0)
    out = hg @ params["fc"]["W"] + params["fc"]["b"]
    return jnp.squeeze(out)
